# baseline (speedup 1.0000x reference)
import functools
import numpy as np
import jax
from jax import lax
import jax.numpy as jnp
from jax.experimental import pallas as pl
from jax.experimental.pallas import tpu as pltpu
from jax.experimental.pallas import tpu_sc as plsc

_B = 128
_D = 3 * 224 * 224
_ALPHA = 0.9
_BETA = 1.0 - _ALPHA

_NC, _NS = 2, 16
_NW = _NC * _NS
_ROWS_PER_W = _B // _NW
_CH = 84
_C = _D // _CH
_GRP = 8
_NGRP = _ROWS_PER_W * _CH // _GRP
_CPW = _ROWS_PER_W * _CH


_PERM = np.asarray([
    98, 105, 103, 43, 22, 94, 86, 125, 49, 0, 45, 108, 56, 121, 62, 109,
    3, 77, 9, 64, 5, 52, 50, 37, 78, 95, 30, 117, 127, 71, 53, 34,
    83, 18, 14, 116, 46, 1, 74, 124, 58, 92, 51, 81, 107, 48, 100, 42,
    106, 8, 69, 101, 90, 110, 66, 65, 21, 17, 67, 4, 32, 102, 27, 33,
    75, 89, 70, 123, 63, 104, 13, 39, 73, 85, 79, 120, 91, 41, 115, 6,
    59, 2, 57, 35, 99, 19, 40, 72, 118, 54, 80, 31, 126, 26, 97, 36,
    38, 25, 47, 61, 96, 15, 28, 68, 60, 82, 112, 55, 44, 119, 11, 114,
    10, 122, 76, 93, 84, 87, 16, 12, 88, 23, 29, 24, 7, 113, 111, 20,
], dtype=np.int32)


def _build_bidx():
    perm = _PERM
    w = np.arange(_NW)[:, None]
    q = np.arange(_CPW)[None, :]
    rows = perm[4 * w + q // _CH]
    return (rows * _CH + q % _CH).astype(np.int32)


_BIDX = _build_bidx()


def _axpy_group(a_ref, b_ref, o_ref):
    def it(t, c):
        for r in range(_GRP):
            for u in range(7):
                sl = pl.ds(t * 112 + u * 16, 16)
                o_ref[r, sl] = _ALPHA * a_ref[r, sl] + _BETA * b_ref[r, sl]
        return c

    lax.fori_loop(0, _C // 112, it, 0)


def _sc_body(x_hbm, bidx_hbm, out_hbm, idx_v,
             a0, a1, b0, b1, o0, o1,
             sa0, sa1, sb0, sb1, so0, so1):
    wid = lax.axis_index("s") * _NC + lax.axis_index("c")
    abase = wid * _CPW
    abufs, bbufs, obufs = (a0, a1), (b0, b1), (o0, o1)
    sas, sbs, sos = (sa0, sa1), (sb0, sb1), (so0, so1)

    pltpu.sync_copy(bidx_hbm.at[wid], idx_v)

    def a_src(s):
        return x_hbm.at[pl.ds(abase + _GRP * s, _GRP)]

    def b_src(s):
        return x_hbm.at[idx_v.at[pl.ds(_GRP * s, _GRP)]]

    def o_dst(s):
        return out_hbm.at[pl.ds(abase + _GRP * s, _GRP)]

    for j in range(2):
        pltpu.make_async_copy(a_src(j), abufs[j], sas[j]).start()
        pltpu.make_async_copy(b_src(j), bbufs[j], sbs[j]).start()

    def step(g, c):
        for j in range(2):
            s = g * 2 + j
            pltpu.make_async_copy(a_src(s), abufs[j], sas[j]).wait()
            pltpu.make_async_copy(b_src(s), bbufs[j], sbs[j]).wait()

            @pl.when(s >= 2)
            def _():
                pltpu.make_async_copy(obufs[j], o_dst(s - 2), sos[j]).wait()

            _axpy_group(abufs[j], bbufs[j], obufs[j])
            pltpu.make_async_copy(obufs[j], o_dst(s), sos[j]).start()

            @pl.when(s < _NGRP - 2)
            def _():
                pltpu.make_async_copy(a_src(s + 2), abufs[j], sas[j]).start()
                pltpu.make_async_copy(b_src(s + 2), bbufs[j], sbs[j]).start()
        return c

    lax.fori_loop(0, _NGRP // 2, step, 0)

    for j in range(2):
        pltpu.make_async_copy(obufs[j], o_dst(_NGRP - 2 + j), sos[j]).wait()


@functools.partial(
    pl.kernel,
    out_type=jax.ShapeDtypeStruct((_B * _CH, _C), jnp.float32),
    mesh=plsc.VectorSubcoreMesh(core_axis_name="c", subcore_axis_name="s"),
    scratch_types=[
        pltpu.VMEM((_CPW,), jnp.int32),
        pltpu.VMEM((_GRP, _C), jnp.float32),
        pltpu.VMEM((_GRP, _C), jnp.float32),
        pltpu.VMEM((_GRP, _C), jnp.float32),
        pltpu.VMEM((_GRP, _C), jnp.float32),
        pltpu.VMEM((_GRP, _C), jnp.float32),
        pltpu.VMEM((_GRP, _C), jnp.float32),
        pltpu.SemaphoreType.DMA,
        pltpu.SemaphoreType.DMA,
        pltpu.SemaphoreType.DMA,
        pltpu.SemaphoreType.DMA,
        pltpu.SemaphoreType.DMA,
        pltpu.SemaphoreType.DMA,
    ],
)
def _mixup_sc(x_hbm, bidx_hbm, out_hbm, *scratch):
    _sc_body(x_hbm, bidx_hbm, out_hbm, *scratch)


def kernel(x):
    x2 = x.reshape(_B * _CH, _C)
    out2 = _mixup_sc(x2, jnp.asarray(_BIDX))
    return out2.reshape(x.shape)


# speedup vs baseline: 2.1353x; 2.1353x over previous
import numpy as np
import jax
import jax.numpy as jnp
from jax.experimental import pallas as pl
from jax.experimental.pallas import tpu as pltpu

_B = 128
_D = 3 * 224 * 224
_DSUB = _D // 128
_ALPHA = 0.9


_PERM = np.asarray([
    98, 105, 103, 43, 22, 94, 86, 125, 49, 0, 45, 108, 56, 121, 62, 109,
    3, 77, 9, 64, 5, 52, 50, 37, 78, 95, 30, 117, 127, 71, 53, 34,
    83, 18, 14, 116, 46, 1, 74, 124, 58, 92, 51, 81, 107, 48, 100, 42,
    106, 8, 69, 101, 90, 110, 66, 65, 21, 17, 67, 4, 32, 102, 27, 33,
    75, 89, 70, 123, 63, 104, 13, 39, 73, 85, 79, 120, 91, 41, 115, 6,
    59, 2, 57, 35, 99, 19, 40, 72, 118, 54, 80, 31, 126, 26, 97, 36,
    38, 25, 47, 61, 96, 15, 28, 68, 60, 82, 112, 55, 44, 119, 11, 114,
    10, 122, 76, 93, 84, 87, 16, 12, 88, 23, 29, 24, 7, 113, 111, 20,
], dtype=np.int32)


def _build_maps():
    perm = _PERM
    seen = np.zeros(_B, dtype=bool)
    order, leader, is_start = [], [], []
    for s in range(_B):
        if seen[s]:
            continue
        j = s
        first = True
        while not seen[j]:
            seen[j] = True
            order.append(j)
            leader.append(s)
            is_start.append(1 if first else 0)
            first = False
            j = int(perm[j])
    order = np.asarray(order, np.int32)
    a_idx = np.asarray(leader, np.int32)
    b_idx = perm[order]
    start = np.asarray(is_start, np.int32)
    return order, a_idx, b_idx, start


_ORDER, _A_IDX, _B_IDX, _START = _build_maps()


def _body(a_map, b_map, o_map, start, a_ref, b_ref, o_ref, prev_ref):
    k = pl.program_id(0)

    @pl.when(start[k] == 1)
    def _():
        prev_ref[...] = a_ref[...]

    o_ref[...] = _ALPHA * prev_ref[...] + (1.0 - _ALPHA) * b_ref[...]
    prev_ref[...] = b_ref[...]


def kernel(x):
    blk = (1, 3, 224, 224)
    grid_spec = pltpu.PrefetchScalarGridSpec(
        num_scalar_prefetch=4,
        grid=(_B,),
        in_specs=[
            pl.BlockSpec(blk, lambda k, a, b, o, s: (a[k], 0, 0, 0)),
            pl.BlockSpec(blk, lambda k, a, b, o, s: (b[k], 0, 0, 0)),
        ],
        out_specs=pl.BlockSpec(blk, lambda k, a, b, o, s: (o[k], 0, 0, 0)),
        scratch_shapes=[pltpu.VMEM(blk, jnp.float32)],
    )
    return pl.pallas_call(
        _body,
        grid_spec=grid_spec,
        out_shape=jax.ShapeDtypeStruct(x.shape, jnp.float32),
    )(
        jnp.asarray(_A_IDX),
        jnp.asarray(_B_IDX),
        jnp.asarray(_ORDER),
        jnp.asarray(_START),
        x,
        x,
    )
